# baseline (speedup 1.0000x reference)
import functools

import jax
import jax.numpy as jnp
from jax import lax
from jax.experimental import pallas as pl
from jax.experimental.pallas import tpu as pltpu
from jax.experimental.pallas import tpu_sc as plsc

_CHUNK = 32
_SCS_ROWS = 1024
_SCS_CHUNK = 512


def _tec_body(nc, rows_per_w, chunk, nchunks, embd_dim, dtype,
              emb_hbm, out_hbm, scs_buf):
    del scs_buf

    def inner(b0, b1, ls0, ls1, ss0, ss1):
        wid = lax.axis_index("s") * nc + lax.axis_index("c")
        base = wid * rows_per_w
        bufs = (b0, b1)
        lsems = (ls0, ls1)
        ssems = (ss0, ss1)

        def src(i):
            return emb_hbm.at[pl.ds(base + i * chunk, chunk)]

        def dst(i):
            return out_hbm.at[pl.ds(base + i * chunk, chunk)]

        loads = {}
        stores = {}
        loads[0] = pltpu.async_copy(src(0), bufs[0], lsems[0])
        if nchunks > 1:
            loads[1] = pltpu.async_copy(src(1), bufs[1], lsems[1])
        for i in range(nchunks):
            b = i % 2
            loads[i].wait()
            stores[i] = pltpu.async_copy(bufs[b], dst(i), ssems[b])
            if i + 2 < nchunks:
                stores[i].wait()
                loads[i + 2] = pltpu.async_copy(src(i + 2), bufs[b], lsems[b])
        for i in range(max(0, nchunks - 2), nchunks):
            stores[i].wait()

    pl.run_scoped(
        inner,
        pltpu.VMEM((chunk, embd_dim), dtype),
        pltpu.VMEM((chunk, embd_dim), dtype),
        pltpu.SemaphoreType.DMA,
        pltpu.SemaphoreType.DMA,
        pltpu.SemaphoreType.DMA,
        pltpu.SemaphoreType.DMA,
    )


def _scs_body(scs_base, rows_per_core, chunk, emb_hbm, out_hbm, scs_buf):
    cid = lax.axis_index("c")
    base = scs_base + cid * rows_per_core
    for j in range(rows_per_core // chunk):
        pltpu.sync_copy(emb_hbm.at[pl.ds(base + j * chunk, chunk)], scs_buf)
        pltpu.sync_copy(scs_buf, out_hbm.at[pl.ds(base + j * chunk, chunk)])


def kernel(inputs, embedding):
    seq_len, embd_dim = embedding.shape
    v_mesh = plsc.VectorSubcoreMesh(core_axis_name="c", subcore_axis_name="s")
    s_mesh = plsc.ScalarSubcoreMesh(axis_name="c", num_cores=v_mesh.num_cores)
    nc = v_mesh.num_cores
    nw = nc * v_mesh.num_subcores
    tec_rows = seq_len - _SCS_ROWS
    rows_per_w = tec_rows // nw
    nchunks = rows_per_w // _CHUNK

    tec_fn = functools.partial(_tec_body, nc, rows_per_w, _CHUNK, nchunks,
                               embd_dim, embedding.dtype)
    scs_fn = functools.partial(_scs_body, tec_rows, _SCS_ROWS // nc,
                               _SCS_CHUNK)
    copy = pl.kernel(
        body=[tec_fn, scs_fn],
        mesh=[v_mesh, s_mesh],
        out_type=jax.ShapeDtypeStruct((seq_len, embd_dim), embedding.dtype),
        scratch_types=[
            pltpu.MemorySpace.VMEM_SHARED((_SCS_CHUNK, embd_dim),
                                          embedding.dtype),
        ],
    )
    out = copy(embedding)
    return out[None]


# speedup vs baseline: 1.0062x; 1.0062x over previous
import functools

import jax
import jax.numpy as jnp
from jax import lax
from jax.experimental import pallas as pl
from jax.experimental.pallas import tpu as pltpu
from jax.experimental.pallas import tpu_sc as plsc

_CHUNK = 32
_SCS_ROWS = 2048
_SCS_CHUNK = 512


def _tec_body(nc, rows_per_w, chunk, nchunks, embd_dim, dtype,
              emb_hbm, out_hbm, scs_buf):
    del scs_buf

    def inner(b0, b1, ls0, ls1, ss0, ss1):
        wid = lax.axis_index("s") * nc + lax.axis_index("c")
        base = wid * rows_per_w
        bufs = (b0, b1)
        lsems = (ls0, ls1)
        ssems = (ss0, ss1)

        def src(i):
            return emb_hbm.at[pl.ds(base + i * chunk, chunk)]

        def dst(i):
            return out_hbm.at[pl.ds(base + i * chunk, chunk)]

        loads = {}
        stores = {}
        loads[0] = pltpu.async_copy(src(0), bufs[0], lsems[0])
        if nchunks > 1:
            loads[1] = pltpu.async_copy(src(1), bufs[1], lsems[1])
        for i in range(nchunks):
            b = i % 2
            loads[i].wait()
            stores[i] = pltpu.async_copy(bufs[b], dst(i), ssems[b])
            if i + 2 < nchunks:
                stores[i].wait()
                loads[i + 2] = pltpu.async_copy(src(i + 2), bufs[b], lsems[b])
        for i in range(max(0, nchunks - 2), nchunks):
            stores[i].wait()

    pl.run_scoped(
        inner,
        pltpu.VMEM((chunk, embd_dim), dtype),
        pltpu.VMEM((chunk, embd_dim), dtype),
        pltpu.SemaphoreType.DMA,
        pltpu.SemaphoreType.DMA,
        pltpu.SemaphoreType.DMA,
        pltpu.SemaphoreType.DMA,
    )


def _scs_body(scs_base, rows_per_core, chunk, emb_hbm, out_hbm, scs_buf):
    cid = lax.axis_index("c")
    base = scs_base + cid * rows_per_core
    for j in range(rows_per_core // chunk):
        pltpu.sync_copy(emb_hbm.at[pl.ds(base + j * chunk, chunk)], scs_buf)
        pltpu.sync_copy(scs_buf, out_hbm.at[pl.ds(base + j * chunk, chunk)])


def kernel(inputs, embedding):
    seq_len, embd_dim = embedding.shape
    v_mesh = plsc.VectorSubcoreMesh(core_axis_name="c", subcore_axis_name="s")
    s_mesh = plsc.ScalarSubcoreMesh(axis_name="c", num_cores=v_mesh.num_cores)
    nc = v_mesh.num_cores
    nw = nc * v_mesh.num_subcores
    tec_rows = seq_len - _SCS_ROWS
    rows_per_w = tec_rows // nw
    nchunks = rows_per_w // _CHUNK

    tec_fn = functools.partial(_tec_body, nc, rows_per_w, _CHUNK, nchunks,
                               embd_dim, embedding.dtype)
    scs_fn = functools.partial(_scs_body, tec_rows, _SCS_ROWS // nc,
                               _SCS_CHUNK)
    copy = pl.kernel(
        body=[tec_fn, scs_fn],
        mesh=[v_mesh, s_mesh],
        out_type=jax.ShapeDtypeStruct((seq_len, embd_dim), embedding.dtype),
        scratch_types=[
            pltpu.MemorySpace.VMEM_SHARED((_SCS_CHUNK, embd_dim),
                                          embedding.dtype),
        ],
    )
    out = copy(embedding)
    return out[None]
